# baseline (speedup 1.0000x reference)
import functools

import jax
import jax.numpy as jnp
from jax import lax
from jax.experimental import pallas as pl
from jax.experimental.pallas import tpu as pltpu
from jax.experimental.pallas import tpu_sc as plsc

_MAX_K = 128
_SEQ = 2048
_D = 64
_T_ROWS = 2 * _MAX_K + 1
_Q_PER_TILE = _SEQ // 32
_PHASES = ((0, 768), (768, 768), (1536, 512))
_WMAX = 768 + _Q_PER_TILE
_RING = 8


def _sc_body(w_hbm, out_hbm, tbuf, wext, sem):
    c = lax.axis_index("c")
    s = lax.axis_index("s")
    wid = s * 2 + c
    q0 = wid * _Q_PER_TILE

    pltpu.sync_copy(w_hbm, tbuf)
    c0 = [tbuf[pl.ds(j * 16, 16)] for j in range(4)]
    cz = [tbuf[pl.ds(256 * _D + j * 16, 16)] for j in range(4)]

    for c0_col, ncol in _PHASES:
        win = ncol + _Q_PER_TILE

        def _drain_one():
            pltpu.make_async_copy(wext.at[pl.ds(0, ncol), :],
                                  out_hbm.at[0, pl.ds(0, ncol), :],
                                  sem).wait()

        lo = c0_col + _SEQ - 1 - (q0 + _Q_PER_TILE - 1)
        p = 1919 - lo
        a = jnp.clip(p, 0, win)
        b = jnp.clip(p + _T_ROWS, 0, win)

        def fill(vj):
            def body(i, _):
                for j in range(4):
                    wext[i, pl.ds(j * 16, 16)] = vj[j]
                return 0
            return body

        def spread(r, _):
            for j in range(4):
                wext[r, pl.ds(j * 16, 16)] = tbuf[pl.ds((r - p) * _D + j * 16,
                                                        16)]
            return 0

        lax.fori_loop(0, a, fill(c0), 0)
        lax.fori_loop(b, win, fill(cz), 0)
        lax.fori_loop(a, b, spread, 0)

        def _start(k):
            src = wext.at[pl.ds(_Q_PER_TILE - 1 - k, ncol), :]
            dst = out_hbm.at[q0 + k, pl.ds(c0_col, ncol), :]
            pltpu.async_copy(src, dst, sem)

        for j in range(_RING):
            _start(j)

        def _steady(k, _):
            _drain_one()
            _start(_RING + k)
            return 0

        lax.fori_loop(0, _Q_PER_TILE - _RING, _steady, 0)
        for j in range(_RING):
            _drain_one()


def kernel(seq_len, emb_weight):
    del seq_len
    mesh = plsc.VectorSubcoreMesh(core_axis_name="c", subcore_axis_name="s")
    run = functools.partial(
        pl.kernel,
        mesh=mesh,
        out_type=jax.ShapeDtypeStruct((_SEQ, _SEQ, _D), jnp.float32),
        scratch_types=[
            pltpu.VMEM((_T_ROWS * _D,), jnp.float32),
            pltpu.VMEM((_WMAX, _D), jnp.float32),
            pltpu.SemaphoreType.DMA,
        ],
        compiler_params=pltpu.CompilerParams(use_tc_tiling_on_sc=True),
    )(_sc_body)
    return run(emb_weight.reshape(-1))
